# baseline (speedup 1.0000x reference)
import functools

import jax
import jax.numpy as jnp
from jax import lax
from jax.experimental import pallas as pl
from jax.experimental.pallas import tpu as pltpu
from jax.experimental.pallas import tpu_sc as plsc

_N = 1000000
_D = 16
_B = 16384

_NC = 2
_NS = 16
_NW = _NC * _NS
_BPW = _B // _NW
_CH = 4
_CHW = _BPW // _CH
_G = _BPW // 16


def _sc_uu_body(w_hbm, idx_i_hbm, idx_j_hbm, out_hbm,
                ii_v, jj_v, wi_v, wj_v, uu_v, sem):
    wid = lax.axis_index("s") * _NC + lax.axis_index("c")
    base = wid * _BPW

    pltpu.sync_copy(idx_i_hbm.at[wid], ii_v)
    pltpu.sync_copy(idx_j_hbm.at[wid], jj_v)

    copies = []
    for c in range(_CH):
        copies.append(pltpu.async_copy(
            w_hbm.at[ii_v.at[c]], wi_v.at[pl.ds(c * _CHW, _CHW)], sem))
        copies.append(pltpu.async_copy(
            w_hbm.at[jj_v.at[c]], wj_v.at[pl.ds(c * _CHW, _CHW)], sem))
    for cp in copies:
        cp.wait()

    def group_body(g, carry):
        rows = g * 16 + lax.iota(jnp.int32, 16)
        sii = jnp.zeros((16,), jnp.float32)
        sjj = jnp.zeros((16,), jnp.float32)
        sdd = jnp.zeros((16,), jnp.float32)
        for d in range(_D):
            cols = jnp.full((16,), d, jnp.int32)
            vi = plsc.load_gather(wi_v, [rows, cols])
            vj = plsc.load_gather(wj_v, [rows, cols])
            diff = vi - vj
            sii = sii + vi * vi
            sjj = sjj + vj * vj
            sdd = sdd + diff * diff
        z = 2.0 * sdd
        denom = (1.0 - sii) * (1.0 - sjj)
        uu = 1.0 + z / denom
        uu_v[pl.ds(g * 16, 16)] = uu
        return carry

    lax.fori_loop(0, _G, group_body, 0)

    pltpu.sync_copy(uu_v, out_hbm.at[pl.ds(base, _BPW)])


@jax.jit
def _sc_uu(w, idx_i, idx_j):
    mesh = plsc.VectorSubcoreMesh(core_axis_name="c", subcore_axis_name="s")
    return pl.kernel(
        _sc_uu_body,
        mesh=mesh,
        compiler_params=pltpu.CompilerParams(
            needs_layout_passes=False, use_tc_tiling_on_sc=False),
        out_type=jax.ShapeDtypeStruct((_B,), jnp.float32),
        scratch_types=[
            pltpu.VMEM((_CH, _CHW), jnp.int32),
            pltpu.VMEM((_CH, _CHW), jnp.int32),
            pltpu.VMEM((_BPW, _D), jnp.float32),
            pltpu.VMEM((_BPW, _D), jnp.float32),
            pltpu.VMEM((_BPW,), jnp.float32),
            pltpu.SemaphoreType.DMA,
        ],
    )(w, idx_i, idx_j)


def _acosh_body(uu_ref, out_ref):
    uu = uu_ref[...]
    out_ref[...] = jnp.log(uu + jnp.sqrt(uu * uu - 1.0))


def _tc_acosh(uu2d):
    return pl.pallas_call(
        _acosh_body,
        out_shape=jax.ShapeDtypeStruct(uu2d.shape, jnp.float32),
    )(uu2d)


def kernel(w, idx):
    idx_i = idx[:, 0].astype(jnp.int32).reshape(_NW, _CH, _CHW)
    idx_j = idx[:, 1].astype(jnp.int32).reshape(_NW, _CH, _CHW)
    uu = _sc_uu(w, idx_i, idx_j)
    d = _tc_acosh(uu.reshape(128, 128)).reshape(_B)
    return d
